# baseline (speedup 1.0000x reference)
import functools

import jax
import jax.numpy as jnp
from jax import lax
from jax.experimental import pallas as pl
from jax.experimental.pallas import tpu as pltpu
from jax.experimental.pallas import tpu_sc as plsc

_VOCAB = 100000
_EMBED = 64
_CTX = 20
_HID = 256
_BATCH = 1024

_CH = 2048
_NFULL = _VOCAB // _CH
_MAIN = _NFULL * _CH
_TAILW = _VOCAB - _MAIN
_NSLOT = 3
_HROWS = _CH // 4


_IDX_CHUNK = 128


def _sc_gather(table, idx):
    info = plsc.get_sparse_core_info()
    nc, ns = info.num_cores, info.num_subcores
    nw = nc * ns
    n = idx.shape[0]
    assert n % (nw * _IDX_CHUNK) == 0
    per_w = n // nw
    chunks = per_w // _IDX_CHUNK
    idx3 = idx.reshape(nw, chunks, _IDX_CHUNK)
    mesh = plsc.VectorSubcoreMesh(core_axis_name="c", subcore_axis_name="s")

    @functools.partial(
        pl.kernel,
        mesh=mesh,
        out_type=jax.ShapeDtypeStruct((n, _EMBED), jnp.float32),
        scratch_types=[
            pltpu.VMEM((chunks, _IDX_CHUNK), jnp.int32),
            pltpu.VMEM((per_w, _EMBED), jnp.float32),
            pltpu.SemaphoreType.DMA,
        ],
        compiler_params=pltpu.CompilerParams(use_tc_tiling_on_sc=False),
    )
    def gather_k(table_hbm, idx_hbm, out_hbm, idx_v, rows_v, sem):
        wid = lax.axis_index("s") * nc + lax.axis_index("c")
        base = wid * per_w
        pltpu.sync_copy(idx_hbm.at[wid], idx_v)
        cps = []
        for i in range(chunks):
            cps.append(pltpu.async_copy(
                table_hbm.at[idx_v.at[i]],
                rows_v.at[pl.ds(i * _IDX_CHUNK, _IDX_CHUNK)],
                sem,
            ))
        for cp in cps:
            cp.wait()
        pltpu.sync_copy(rows_v, out_hbm.at[pl.ds(base, per_w)])

    return gather_k(table, idx3)


_TDIMS = (((0,), (0,)), ((), ()))
_TDIMS_X = (((0,), (1,)), ((), ()))


def _mega_body(x_ref, w1_ref, b1c_ref, b2m_ref, b2t_ref, w2t_ref, w2_hbm,
               logits_hbm, probas_hbm,
               w2_buf, out_buf, tail_buf, in_sem, out_sem, tail_sem):
    ht = jnp.maximum(
        lax.dot_general(w1_ref[...], x_ref[...], _TDIMS_X,
                        preferred_element_type=jnp.float32)
        + b1c_ref[...], 0.0)

    def in_cp(j, slot):
        return pltpu.make_async_copy(
            w2_hbm.at[:, pl.ds(pl.multiple_of(j * _CH, _CH), _CH)],
            w2_buf.at[slot],
            in_sem.at[slot])

    def out_cp(hbm, j, slot, q):
        return pltpu.make_async_copy(
            out_buf.at[slot, pl.ds(q * _HROWS, _HROWS)],
            hbm.at[pl.ds(j * _CH + q * _HROWS, _HROWS)],
            out_sem.at[slot, q])

    def tail_cp(hbm, half):
        return pltpu.make_async_copy(
            tail_buf.at[pl.ds(half * (_TAILW // 2), _TAILW // 2)],
            hbm.at[pl.ds(_MAIN + half * (_TAILW // 2), _TAILW // 2)],
            tail_sem.at[half])

    def sweep(out_hbm, stats, fin, carry0):
        for r in range(_NSLOT):
            in_cp(r, r).start()
        ttile = lax.dot_general(w2t_ref[...], ht, _TDIMS,
                                preferred_element_type=jnp.float32)
        ttile = fin(ttile + b2t_ref[...])
        tail_buf[...] = ttile
        tail_cp(out_hbm, 0).start()
        tail_cp(out_hbm, 1).start()
        if stats:
            m0 = jnp.max(ttile, axis=0, keepdims=True)
            s0 = jnp.sum(jnp.exp(ttile - m0), axis=0, keepdims=True)
            carry0 = (m0, s0)

        def step(k, carry):
            for r in range(_NSLOT):
                j = k * _NSLOT + r
                in_cp(j, r).wait()
                tile = lax.dot_general(w2_buf[r, :, :], ht, _TDIMS,
                                       preferred_element_type=jnp.float32)
                tile = fin(tile + jnp.transpose(b2m_ref[pl.ds(j, 1), :]))

                @pl.when(k >= 1)
                def _():
                    for q in range(4):
                        out_cp(out_hbm, j, r, q).wait()

                out_buf[r, :, :] = tile
                for q in range(4):
                    out_cp(out_hbm, j, r, q).start()

                @pl.when(k < _NFULL // _NSLOT - 1)
                def _():
                    in_cp(j + _NSLOT, r).start()

                if stats:
                    m, s = carry
                    tmax = jnp.max(tile, axis=0, keepdims=True)
                    m_new = jnp.maximum(m, tmax)
                    s = (s * jnp.exp(m - m_new)
                         + jnp.sum(jnp.exp(tile - m_new), axis=0,
                                   keepdims=True))
                    carry = (m_new, s)
            return carry

        carry = lax.fori_loop(0, _NFULL // _NSLOT, step, carry0)
        return carry

    def drain(out_hbm):
        for r in range(_NSLOT):
            for q in range(4):
                out_cp(out_hbm, _NFULL - _NSLOT + r, r, q).wait()
        tail_cp(out_hbm, 0).wait()
        tail_cp(out_hbm, 1).wait()

    m, s = sweep(logits_hbm, True, lambda t: t, None)
    logz = m + jnp.log(s)
    drain(logits_hbm)
    sweep(probas_hbm, False, lambda t: t - logz, 0)
    drain(probas_hbm)


def _mega(x, w1, b1c, b2m, b2t, w2t, w2):
    return pl.pallas_call(
        _mega_body,
        in_specs=[
            pl.BlockSpec(memory_space=pltpu.MemorySpace.VMEM),
            pl.BlockSpec(memory_space=pltpu.MemorySpace.VMEM),
            pl.BlockSpec(memory_space=pltpu.MemorySpace.VMEM),
            pl.BlockSpec(memory_space=pltpu.MemorySpace.VMEM),
            pl.BlockSpec(memory_space=pltpu.MemorySpace.VMEM),
            pl.BlockSpec(memory_space=pltpu.MemorySpace.VMEM),
            pl.BlockSpec(memory_space=pltpu.MemorySpace.HBM),
        ],
        out_specs=[
            pl.BlockSpec(memory_space=pltpu.MemorySpace.HBM),
            pl.BlockSpec(memory_space=pltpu.MemorySpace.HBM),
        ],
        out_shape=[
            jax.ShapeDtypeStruct((_VOCAB, _BATCH), jnp.float32),
            jax.ShapeDtypeStruct((_VOCAB, _BATCH), jnp.float32),
        ],
        scratch_shapes=[
            pltpu.VMEM((_NSLOT, _HID, _CH), jnp.float32),
            pltpu.VMEM((_NSLOT, _CH, _BATCH), jnp.float32),
            pltpu.VMEM((_TAILW, _BATCH), jnp.float32),
            pltpu.SemaphoreType.DMA((_NSLOT,)),
            pltpu.SemaphoreType.DMA((_NSLOT, 4)),
            pltpu.SemaphoreType.DMA((2,)),
        ],
        compiler_params=pltpu.CompilerParams(
            vmem_limit_bytes=63 * 1024 * 1024),
    )(x, w1, b1c, b2m, b2t, w2t, w2)


def kernel(inputs, embed_table, W1, b1, W2, b2):
    idx = inputs.reshape(-1).astype(jnp.int32)
    x = _sc_gather(embed_table, idx)
    x = x.reshape(_BATCH, _CTX * _EMBED)
    b1c = b1.reshape(_HID, 1)
    b2m = b2[:_MAIN].reshape(_NFULL, _CH)
    b2t = b2[_MAIN:].reshape(_TAILW, 1)
    w2t = W2[:, _MAIN:]
    logitsT, probasT = _mega(x, W1, b1c, b2m, b2t, w2t, W2)
    return (logitsT.T, probasT.T)
